# baseline (speedup 1.0000x reference)
import jax
import jax.numpy as jnp
from jax.experimental import pallas as pl
from jax.experimental.pallas import tpu as pltpu

N = 10000
D_IN = 128
D_OUT = 128
NEG_SLOPE = 0.2
BM = 400
NSTEP = N // BM
NBUF = 3


def _body(x_hbm, w_hbm, adj_hbm, out_hbm,
          x_v, w_v, sup, abuf, obuf,
          sem_x, sem_w, sem_a, sem_o):
    pltpu.make_async_copy(x_hbm, x_v, sem_x).start()
    pltpu.make_async_copy(w_hbm, w_v, sem_w).start()
    for s in range(NBUF):
        pltpu.make_async_copy(
            adj_hbm.at[pl.ds(s * BM, BM), :], abuf.at[s], sem_a.at[s]
        ).start()
    pltpu.make_async_copy(x_hbm, x_v, sem_x).wait()
    pltpu.make_async_copy(w_hbm, w_v, sem_w).wait()
    sup[...] = jnp.dot(x_v[...], w_v[...], preferred_element_type=jnp.float32)

    def step(i, carry):
        slot = jax.lax.rem(i, NBUF)
        oslot = jax.lax.rem(i, 2)
        pltpu.make_async_copy(
            adj_hbm.at[pl.ds(i * BM, BM), :], abuf.at[slot], sem_a.at[slot]
        ).wait()
        acc = jnp.dot(abuf[slot], sup[...], preferred_element_type=jnp.float32)

        @pl.when(i >= 2)
        def _():
            pltpu.make_async_copy(
                obuf.at[oslot],
                out_hbm.at[pl.ds((i - 2) * BM, BM), :],
                sem_o.at[oslot],
            ).wait()

        obuf[oslot] = jnp.where(acc >= 0, acc, NEG_SLOPE * acc)
        pltpu.make_async_copy(
            obuf.at[oslot], out_hbm.at[pl.ds(i * BM, BM), :], sem_o.at[oslot]
        ).start()

        @pl.when(i + NBUF < NSTEP)
        def _():
            pltpu.make_async_copy(
                adj_hbm.at[pl.ds((i + NBUF) * BM, BM), :],
                abuf.at[slot],
                sem_a.at[slot],
            ).start()

        return carry

    jax.lax.fori_loop(0, NSTEP, step, 0)
    pltpu.make_async_copy(
        obuf.at[(NSTEP - 2) % 2],
        out_hbm.at[pl.ds((NSTEP - 2) * BM, BM), :],
        sem_o.at[(NSTEP - 2) % 2],
    ).wait()
    pltpu.make_async_copy(
        obuf.at[(NSTEP - 1) % 2],
        out_hbm.at[pl.ds((NSTEP - 1) * BM, BM), :],
        sem_o.at[(NSTEP - 1) % 2],
    ).wait()


@jax.jit
def kernel(input_features, adj, W):
    return pl.pallas_call(
        _body,
        in_specs=[
            pl.BlockSpec(memory_space=pltpu.HBM),
            pl.BlockSpec(memory_space=pltpu.HBM),
            pl.BlockSpec(memory_space=pltpu.HBM),
        ],
        out_specs=pl.BlockSpec(memory_space=pltpu.HBM),
        out_shape=jax.ShapeDtypeStruct((N, D_OUT), jnp.float32),
        scratch_shapes=[
            pltpu.VMEM((N, D_IN), jnp.float32),
            pltpu.VMEM((D_IN, D_OUT), jnp.float32),
            pltpu.VMEM((N, D_OUT), jnp.float32),
            pltpu.VMEM((NBUF, BM, N), jnp.float32),
            pltpu.VMEM((2, BM, D_OUT), jnp.float32),
            pltpu.SemaphoreType.DMA,
            pltpu.SemaphoreType.DMA,
            pltpu.SemaphoreType.DMA((NBUF,)),
            pltpu.SemaphoreType.DMA((2,)),
        ],
        compiler_params=pltpu.CompilerParams(
            vmem_limit_bytes=64 * 1024 * 1024,
        ),
    )(input_features, W, adj)


# speedup vs baseline: 1.0259x; 1.0259x over previous
import jax
import jax.numpy as jnp
from jax.experimental import pallas as pl
from jax.experimental.pallas import tpu as pltpu

N = 10000
D_IN = 128
D_OUT = 128
NEG_SLOPE = 0.2
BM = 400


def _fused_body(x_ref, w_ref, adj_ref, out_ref, sup_ref):
    @pl.when(pl.program_id(0) == 0)
    def _():
        sup_ref[...] = jnp.dot(
            x_ref[...], w_ref[...], preferred_element_type=jnp.float32
        )

    acc = jnp.dot(adj_ref[...], sup_ref[...], preferred_element_type=jnp.float32)
    out_ref[...] = jnp.where(acc >= 0, acc, NEG_SLOPE * acc)


@jax.jit
def kernel(input_features, adj, W):
    return pl.pallas_call(
        _fused_body,
        grid=(N // BM,),
        in_specs=[
            pl.BlockSpec((N, D_IN), lambda i: (0, 0)),
            pl.BlockSpec((D_IN, D_OUT), lambda i: (0, 0)),
            pl.BlockSpec((BM, N), lambda i: (i, 0)),
        ],
        out_specs=pl.BlockSpec((BM, D_OUT), lambda i: (i, 0)),
        out_shape=jax.ShapeDtypeStruct((N, D_OUT), jnp.float32),
        scratch_shapes=[pltpu.VMEM((N, D_OUT), jnp.float32)],
        compiler_params=pltpu.CompilerParams(
            dimension_semantics=("arbitrary",),
        ),
    )(input_features, W, adj)


# speedup vs baseline: 1.0366x; 1.0105x over previous
import jax
import jax.numpy as jnp
from jax.experimental import pallas as pl
from jax.experimental.pallas import tpu as pltpu

N = 10000
D_IN = 128
D_OUT = 128
NEG_SLOPE = 0.2
BM = 400


def _fused_body(adj_ref, x_ref, w_ref, out_ref, sup_ref):
    @pl.when(pl.program_id(0) == 0)
    def _():
        sup_ref[...] = jnp.dot(
            x_ref[...], w_ref[...], preferred_element_type=jnp.float32
        )

    acc = jnp.dot(adj_ref[...], sup_ref[...], preferred_element_type=jnp.float32)
    out_ref[...] = jnp.where(acc >= 0, acc, NEG_SLOPE * acc)


@jax.jit
def kernel(input_features, adj, W):
    return pl.pallas_call(
        _fused_body,
        grid=(N // BM,),
        in_specs=[
            pl.BlockSpec((BM, N), lambda i: (i, 0)),
            pl.BlockSpec((N, D_IN), lambda i: (0, 0)),
            pl.BlockSpec((D_IN, D_OUT), lambda i: (0, 0)),
        ],
        out_specs=pl.BlockSpec((BM, D_OUT), lambda i: (i, 0)),
        out_shape=jax.ShapeDtypeStruct((N, D_OUT), jnp.float32),
        scratch_shapes=[pltpu.VMEM((N, D_OUT), jnp.float32)],
        compiler_params=pltpu.CompilerParams(
            dimension_semantics=("arbitrary",),
        ),
    )(adj, input_features, W)
